# baseline (speedup 1.0000x reference)
import math

import jax
import jax.numpy as jnp
from jax import lax
from jax.experimental import pallas as pl
from jax.experimental.pallas import tpu as pltpu

H = 256
F = 8
TB = 4096


def _body(a_ref, e0, e1, e2, e3, e4, e5, out_ref):
    s = math.sqrt(H)
    es = [e0[...], e1[...], e2[...], e3[...], e4[...], e5[...]]
    deltas = [(e[1:2, :] - e[0:1, :]) * s for e in es]
    base = (es[0][0:1] + es[1][0:1] + es[2][0:1]
            + es[3][0:1] + es[4][0:1] + es[5][0:1]) * s
    d = jnp.concatenate(deltas + [base, jnp.zeros_like(base)], axis=0)
    x = a_ref[...].astype(jnp.float32)
    out_ref[...] = lax.dot_general(
        x, d, dimension_numbers=(((0,), (0,)), ((), ())),
        preferred_element_type=jnp.float32)


def kernel(mix, falsetto, breathy, pharyngeal, glissando, vibrato,
           mix_emb, falsetto_emb, breathy_emb, pharyngeal_emb,
           glissando_emb, vibrato_emb):
    B, L = mix.shape
    n = B * L
    flat = [a.reshape(n) for a in
            (mix, falsetto, breathy, pharyngeal, glissando, vibrato)]
    a = jnp.stack(flat + [jnp.ones((n,), jnp.int32),
                          jnp.zeros((n,), jnp.int32)], axis=0)
    embs = (mix_emb, falsetto_emb, breathy_emb, pharyngeal_emb,
            glissando_emb, vibrato_emb)
    grid = (n // TB,)
    emb_spec = pl.BlockSpec((2, H), lambda i: (0, 0))
    out = pl.pallas_call(
        _body,
        grid=grid,
        in_specs=[pl.BlockSpec((F, TB), lambda i: (0, i))]
        + [emb_spec] * 6,
        out_specs=pl.BlockSpec((TB, H), lambda i: (i, 0)),
        out_shape=jax.ShapeDtypeStruct((n, H), jnp.float32),
    )(a, *embs)
    return out.reshape(B, L, H)


# speedup vs baseline: 1.0746x; 1.0746x over previous
import math

import jax
import jax.numpy as jnp
from jax import lax
from jax.experimental import pallas as pl
from jax.experimental.pallas import tpu as pltpu

H = 256
F = 8
TB = 20480


def _body(a_ref, e0, e1, e2, e3, e4, e5, out_ref):
    s = math.sqrt(H)
    es = [e0[...], e1[...], e2[...], e3[...], e4[...], e5[...]]
    deltas = [(e[1:2, :] - e[0:1, :]) * s for e in es]
    base = (es[0][0:1] + es[1][0:1] + es[2][0:1]
            + es[3][0:1] + es[4][0:1] + es[5][0:1]) * s
    d = jnp.concatenate(deltas + [base, jnp.zeros_like(base)], axis=0)
    x = a_ref[...].astype(jnp.float32)
    out_ref[...] = lax.dot_general(
        x, d, dimension_numbers=(((0,), (0,)), ((), ())),
        preferred_element_type=jnp.float32)


def kernel(mix, falsetto, breathy, pharyngeal, glissando, vibrato,
           mix_emb, falsetto_emb, breathy_emb, pharyngeal_emb,
           glissando_emb, vibrato_emb):
    B, L = mix.shape
    n = B * L
    flat = [a.reshape(n) for a in
            (mix, falsetto, breathy, pharyngeal, glissando, vibrato)]
    a = jnp.stack(flat + [jnp.ones((n,), jnp.int32),
                          jnp.zeros((n,), jnp.int32)], axis=0)
    embs = (mix_emb, falsetto_emb, breathy_emb, pharyngeal_emb,
            glissando_emb, vibrato_emb)
    grid = (n // TB,)
    emb_spec = pl.BlockSpec((2, H), lambda i: (0, 0))
    out = pl.pallas_call(
        _body,
        grid=grid,
        in_specs=[pl.BlockSpec((F, TB), lambda i: (0, i))]
        + [emb_spec] * 6,
        out_specs=pl.BlockSpec((TB, H), lambda i: (i, 0)),
        out_shape=jax.ShapeDtypeStruct((n, H), jnp.float32),
    )(a, *embs)
    return out.reshape(B, L, H)


# speedup vs baseline: 1.0920x; 1.0161x over previous
import math

import jax
import jax.numpy as jnp
from jax import lax
from jax.experimental import pallas as pl
from jax.experimental.pallas import tpu as pltpu

H = 256
F = 8
TB = 10240


def _body(a_ref, e0, e1, e2, e3, e4, e5, out_ref):
    s = math.sqrt(H)
    es = [e0[...], e1[...], e2[...], e3[...], e4[...], e5[...]]
    deltas = [(e[1:2, :] - e[0:1, :]) * s for e in es]
    base = (es[0][0:1] + es[1][0:1] + es[2][0:1]
            + es[3][0:1] + es[4][0:1] + es[5][0:1]) * s
    d = jnp.concatenate(deltas + [base, jnp.zeros_like(base)], axis=0)
    x = a_ref[...].astype(jnp.float32)
    out_ref[...] = lax.dot_general(
        x, d, dimension_numbers=(((0,), (0,)), ((), ())),
        preferred_element_type=jnp.float32)


def kernel(mix, falsetto, breathy, pharyngeal, glissando, vibrato,
           mix_emb, falsetto_emb, breathy_emb, pharyngeal_emb,
           glissando_emb, vibrato_emb):
    B, L = mix.shape
    n = B * L
    flat = [a.reshape(n) for a in
            (mix, falsetto, breathy, pharyngeal, glissando, vibrato)]
    a = jnp.stack(flat + [jnp.ones((n,), jnp.int32),
                          jnp.zeros((n,), jnp.int32)], axis=0)
    embs = (mix_emb, falsetto_emb, breathy_emb, pharyngeal_emb,
            glissando_emb, vibrato_emb)
    grid = (n // TB,)
    emb_spec = pl.BlockSpec((2, H), lambda i: (0, 0))
    out = pl.pallas_call(
        _body,
        grid=grid,
        in_specs=[pl.BlockSpec((F, TB), lambda i: (0, i))]
        + [emb_spec] * 6,
        out_specs=pl.BlockSpec((TB, H), lambda i: (i, 0)),
        out_shape=jax.ShapeDtypeStruct((n, H), jnp.float32),
    )(a, *embs)
    return out.reshape(B, L, H)
